# baseline (speedup 1.0000x reference)
import functools

import jax
import jax.numpy as jnp
from jax import lax
from jax.experimental import pallas as pl
from jax.experimental.pallas import tpu as pltpu
from jax.experimental.pallas import tpu_sc as plsc

N_NODES = 10000
D_FEAT = 256
D_HALF = 128
D_HID = 128
N_ADJ = 160000
N_LINK = 20000

NC = 2
NS = 16
NW = NC * NS

_KB1 = 80
_EPW = _KB1 * 128
_EPAD = NS * _EPW
_ROWS1 = 2 * _EPAD // 128
_ACC_ROWS = 10112
_CH = 40

_KB3 = 5
_PPW = _KB3 * 128
_LPAD = NW * _PPW

def _p1_body(xi, srcr, dstr, z2, z1, o1, agg_out, deg0_out, deg1_out,
             acc_sh, deg_sh, src_v, dst_v, buf_a, buf_b, ones_v, deg_buf,
             sem_a, sem_b, sem_d):
    c = lax.axis_index("c")
    s = lax.axis_index("s")
    w = c * NS + s

    pltpu.sync_copy(o1, ones_v)

    rz = _ACC_ROWS // NS
    pltpu.sync_copy(z2.at[pl.ds(s * rz, rz)], acc_sh.at[pl.ds(s * rz, rz)])
    pltpu.sync_copy(z1.at[pl.ds(s * rz, rz)], deg_buf.at[pl.ds(0, rz)])
    pltpu.sync_copy(deg_buf.at[pl.ds(0, rz)], deg_sh.at[pl.ds(s * rz, rz)])

    plsc.subcore_barrier()

    bufs = (buf_a, buf_b)
    sems = (sem_a, sem_b)

    for hp in range(_KB1 // _CH):
        base = w * _KB1 + hp * _CH
        pltpu.sync_copy(srcr.at[pl.ds(base, _CH)], src_v)
        pltpu.sync_copy(dstr.at[pl.ds(base, _CH)], dst_v)
        pltpu.async_copy(xi.at[src_v.at[0]], bufs[0], sems[0])
        deg_on = c == hp

        def halfpass(i, carry):
            for p in range(2):
                b = 2 * i + p
                nxt = jnp.minimum(b + 1, _CH - 1)
                pltpu.async_copy(xi.at[src_v.at[nxt]], bufs[(p + 1) % 2],
                                 sems[(p + 1) % 2])
                pltpu.make_async_copy(xi.at[src_v.at[0]], bufs[p], sems[p]).wait()
                pltpu.sync_copy(bufs[p], acc_sh.at[dst_v.at[b]], add=True)

                @pl.when(deg_on)
                def _():
                    pltpu.async_copy(ones_v, deg_sh.at[dst_v.at[b]], sem_d,
                                     add=True)

            return carry

        lax.fori_loop(0, _CH // 2, halfpass, 0)
        pltpu.make_async_copy(xi.at[src_v.at[0]], bufs[0], sems[0]).wait()

        @pl.when(deg_on)
        def _():
            for _i in range(_CH):
                pltpu.make_async_copy(ones_v, deg_sh.at[dst_v.at[0]], sem_d).wait()

    plsc.subcore_barrier()

    ro = _ACC_ROWS // NS
    pltpu.sync_copy(acc_sh.at[pl.ds(s * ro, ro)], agg_out.at[c, pl.ds(s * ro, ro)])

    pltpu.sync_copy(deg_sh.at[pl.ds(s * rz, rz)], deg_buf.at[pl.ds(0, rz)])

    @pl.when(c == 0)
    def _():
        pltpu.sync_copy(deg_buf.at[pl.ds(0, rz)], deg0_out.at[pl.ds(s * rz, rz)])

    @pl.when(c == 1)
    def _():
        pltpu.sync_copy(deg_buf.at[pl.ds(0, rz)], deg1_out.at[pl.ds(s * rz, rz)])


@functools.lru_cache(maxsize=None)
def _make_p1():
    mesh = plsc.VectorSubcoreMesh(
        core_axis_name="c", subcore_axis_name="s", num_cores=NC, num_subcores=NS)
    return pl.kernel(
        _p1_body,
        out_type=(jax.ShapeDtypeStruct((NC, _ACC_ROWS, D_HALF), jnp.float32),
                  jax.ShapeDtypeStruct((_ACC_ROWS,), jnp.float32),
                  jax.ShapeDtypeStruct((_ACC_ROWS,), jnp.float32)),
        mesh=mesh,
        scratch_types=[
            pltpu.VMEM_SHARED((_ACC_ROWS, D_HALF), jnp.float32),
            pltpu.VMEM_SHARED((_ACC_ROWS,), jnp.float32),
            pltpu.VMEM((_CH, 128), jnp.int32),
            pltpu.VMEM((_CH, 128), jnp.int32),
            pltpu.VMEM((128, D_HALF), jnp.float32),
            pltpu.VMEM((128, D_HALF), jnp.float32),
            pltpu.VMEM((128,), jnp.float32),
            pltpu.VMEM((_ACC_ROWS // NS,), jnp.float32),
            pltpu.SemaphoreType.DMA,
            pltpu.SemaphoreType.DMA,
            pltpu.SemaphoreType.DMA,
        ])


def _p3_body(h, e0r, e1r, hi_out, hj_out, e0_v, e1_v, buf_i, buf_j, sem_i, sem_j):
    c = lax.axis_index("c")
    s = lax.axis_index("s")
    w = c * NS + s
    pltpu.sync_copy(e0r, e0_v)
    pltpu.sync_copy(e1r, e1_v)
    for b in range(_KB3):
        row = w * _KB3 + b
        cp_i = pltpu.async_copy(h.at[e0_v.at[row]], buf_i, sem_i)
        cp_j = pltpu.async_copy(h.at[e1_v.at[row]], buf_j, sem_j)
        cp_i.wait()
        cp_j.wait()
        base = w * _PPW + b * 128
        pltpu.sync_copy(buf_i, hi_out.at[pl.ds(base, 128)])
        pltpu.sync_copy(buf_j, hj_out.at[pl.ds(base, 128)])


@functools.lru_cache(maxsize=None)
def _make_p3():
    mesh = plsc.VectorSubcoreMesh(
        core_axis_name="c", subcore_axis_name="s", num_cores=NC, num_subcores=NS)
    return pl.kernel(
        _p3_body,
        out_type=(jax.ShapeDtypeStruct((_LPAD, D_FEAT), jnp.float32),
                  jax.ShapeDtypeStruct((_LPAD, D_FEAT), jnp.float32)),
        mesh=mesh,
        scratch_types=[
            pltpu.VMEM((_LPAD // 128, 128), jnp.int32),
            pltpu.VMEM((_LPAD // 128, 128), jnp.int32),
            pltpu.VMEM((128, D_FEAT), jnp.float32),
            pltpu.VMEM((128, D_FEAT), jnp.float32),
            pltpu.SemaphoreType.DMA,
            pltpu.SemaphoreType.DMA,
        ])


def _gnn_mm(a0_ref, a1_ref, d0_ref, d1_ref, w0_ref, w1_ref, b_ref, out_ref):
    r = 1.0 / jnp.maximum(d0_ref[...] + d1_ref[...], 1.0)
    acc = jnp.dot(a0_ref[0] * r, w0_ref[0], preferred_element_type=jnp.float32)
    acc += jnp.dot(a1_ref[0] * r, w1_ref[0], preferred_element_type=jnp.float32)
    out_ref[...] = jnp.maximum(acc + b_ref[...], 0.0)


def _mlp(hi_ref, hj_ref, w1_ref, b1_ref, w2_ref, b2_ref, out_ref):
    z = hi_ref[...] * hj_ref[...]
    t = jnp.dot(z, w1_ref[...], preferred_element_type=jnp.float32) + b1_ref[...]
    t = jnp.maximum(t, 0.0)
    logit = jnp.sum(t * w2_ref[...], axis=1, keepdims=True) + b2_ref[...]
    out_ref[...] = 1.0 / (1.0 + jnp.exp(-logit))


def kernel(x, edges, adj, W_gnn, b_gnn, W1, b1, W2, b2):
    x = x.astype(jnp.float32)
    src = adj[0].astype(jnp.int32)
    dst = adj[1].astype(jnp.int32)
    e0 = edges[0].astype(jnp.int32)
    e1 = edges[1].astype(jnp.int32)

    xi = x.reshape(N_NODES, 2, D_HALF).reshape(2 * N_NODES, D_HALF)
    pad = _EPAD - N_ADJ
    src_p = jnp.concatenate([src, jnp.zeros((pad,), jnp.int32)])
    dst_p = jnp.concatenate([dst, jnp.full((pad,), N_NODES, jnp.int32)])
    src_all = jnp.concatenate([2 * src_p, 2 * src_p + 1]).reshape(_ROWS1, 128)
    dst_all = jnp.concatenate([dst_p, dst_p]).reshape(_ROWS1, 128)
    z2 = jnp.zeros((_ACC_ROWS, D_HALF), jnp.float32)
    z1 = jnp.zeros((_ACC_ROWS,), jnp.float32)
    o1 = jnp.ones((128,), jnp.float32)

    agg, deg0, deg1 = _make_p1()(xi, src_all, dst_all, z2, z1, o1)
    d0 = deg0.reshape(_ACC_ROWS, 1)
    d1 = deg1.reshape(_ACC_ROWS, 1)

    bm = 1000
    h = pl.pallas_call(
        _gnn_mm,
        grid=(N_NODES // bm,),
        in_specs=[
            pl.BlockSpec((1, bm, D_HALF), lambda i: (0, i, 0)),
            pl.BlockSpec((1, bm, D_HALF), lambda i: (1, i, 0)),
            pl.BlockSpec((bm, 1), lambda i: (i, 0)),
            pl.BlockSpec((bm, 1), lambda i: (i, 0)),
            pl.BlockSpec((1, D_HALF, D_FEAT), lambda i: (0, 0, 0)),
            pl.BlockSpec((1, D_HALF, D_FEAT), lambda i: (1, 0, 0)),
            pl.BlockSpec((1, D_FEAT), lambda i: (0, 0)),
        ],
        out_specs=pl.BlockSpec((bm, D_FEAT), lambda i: (i, 0)),
        out_shape=jax.ShapeDtypeStruct((N_NODES, D_FEAT), jnp.float32),
    )(agg, agg, d0, d1, W_gnn.reshape(2, D_HALF, D_FEAT),
      W_gnn.reshape(2, D_HALF, D_FEAT), b_gnn.reshape(1, D_FEAT))

    lpad = _LPAD - N_LINK
    e0_p = jnp.concatenate([e0, jnp.zeros((lpad,), jnp.int32)]).reshape(_LPAD // 128, 128)
    e1_p = jnp.concatenate([e1, jnp.zeros((lpad,), jnp.int32)]).reshape(_LPAD // 128, 128)
    hi, hj = _make_p3()(h, e0_p, e1_p)

    bl = 1024
    logits = pl.pallas_call(
        _mlp,
        grid=(_LPAD // bl,),
        in_specs=[
            pl.BlockSpec((bl, D_FEAT), lambda i: (i, 0)),
            pl.BlockSpec((bl, D_FEAT), lambda i: (i, 0)),
            pl.BlockSpec((D_FEAT, D_HID), lambda i: (0, 0)),
            pl.BlockSpec((1, D_HID), lambda i: (0, 0)),
            pl.BlockSpec((1, D_HID), lambda i: (0, 0)),
            pl.BlockSpec((1, 1), lambda i: (0, 0)),
        ],
        out_specs=pl.BlockSpec((bl, 1), lambda i: (i, 0)),
        out_shape=jax.ShapeDtypeStruct((_LPAD, 1), jnp.float32),
    )(hi, hj, W1, b1.reshape(1, D_HID), W2.reshape(1, D_HID), b2.reshape(1, 1))

    return logits[:N_LINK, 0]


# speedup vs baseline: 1.0125x; 1.0125x over previous
import functools

import jax
import jax.numpy as jnp
from jax import lax
from jax.experimental import pallas as pl
from jax.experimental.pallas import tpu as pltpu
from jax.experimental.pallas import tpu_sc as plsc

N_NODES = 10000
D_FEAT = 256
D_HALF = 128
D_HID = 128
N_ADJ = 160000
N_LINK = 20000

NC = 2
NS = 16
NW = NC * NS

_KB1 = 80
_EPW = _KB1 * 128
_EPAD = NS * _EPW
_ROWS1 = 2 * _EPAD // 128
_ACC_ROWS = 10112
_CH = 40

_KB3 = 5
_PPW = _KB3 * 128
_LPAD = NW * _PPW

def _p1_body(xi, srcr, dstr, z2, z1, o1, agg_out, deg0_out, deg1_out,
             acc_sh, deg_sh, src_v, dst_v, buf_a, buf_b, ones_v, deg_buf,
             sem_a, sem_b, sem_s, sem_d):
    c = lax.axis_index("c")
    s = lax.axis_index("s")
    w = c * NS + s

    pltpu.sync_copy(o1, ones_v)

    rz = _ACC_ROWS // NS
    pltpu.sync_copy(z2.at[pl.ds(s * rz, rz)], acc_sh.at[pl.ds(s * rz, rz)])
    pltpu.sync_copy(z1.at[pl.ds(s * rz, rz)], deg_buf.at[pl.ds(0, rz)])
    pltpu.sync_copy(deg_buf.at[pl.ds(0, rz)], deg_sh.at[pl.ds(s * rz, rz)])

    plsc.subcore_barrier()

    bufs = (buf_a, buf_b)
    sems = (sem_a, sem_b)

    def wait_gather(b_par):
        pltpu.make_async_copy(xi.at[src_v.at[0]], bufs[b_par], sems[b_par]).wait()

    def issue_scatter(b_par, b):
        pltpu.async_copy(bufs[b_par], acc_sh.at[dst_v.at[b]], sem_s, add=True)

    def wait_scatter(b_par):
        pltpu.make_async_copy(bufs[b_par], acc_sh.at[dst_v.at[0]], sem_s).wait()

    for hp in range(_KB1 // _CH):
        base = w * _KB1 + hp * _CH
        pltpu.sync_copy(srcr.at[pl.ds(base, _CH)], src_v)
        pltpu.sync_copy(dstr.at[pl.ds(base, _CH)], dst_v)
        deg_on = c == hp

        def deg_scatter(b):
            @pl.when(deg_on)
            def _():
                pltpu.async_copy(ones_v, deg_sh.at[dst_v.at[b]], sem_d,
                                 add=True)

        pltpu.async_copy(xi.at[src_v.at[0]], buf_a, sem_a)
        pltpu.async_copy(xi.at[src_v.at[1]], buf_b, sem_b)
        wait_gather(0)
        issue_scatter(0, 0)
        deg_scatter(0)

        def halfpass(i, carry):
            b1 = 2 * i + 1
            wait_scatter(0)
            pltpu.async_copy(xi.at[src_v.at[b1 + 1]], buf_a, sem_a)
            wait_gather(1)
            issue_scatter(1, b1)
            deg_scatter(b1)

            b2 = b1 + 1
            wait_scatter(1)
            pltpu.async_copy(xi.at[src_v.at[b2 + 1]], buf_b, sem_b)
            wait_gather(0)
            issue_scatter(0, b2)
            deg_scatter(b2)
            return carry

        lax.fori_loop(0, _CH // 2 - 1, halfpass, 0)

        wait_scatter(0)
        wait_gather(1)
        issue_scatter(1, _CH - 1)
        deg_scatter(_CH - 1)
        wait_scatter(1)

        @pl.when(deg_on)
        def _():
            for _i in range(_CH):
                pltpu.make_async_copy(ones_v, deg_sh.at[dst_v.at[0]], sem_d).wait()

    plsc.subcore_barrier()

    ro = _ACC_ROWS // NS
    pltpu.sync_copy(acc_sh.at[pl.ds(s * ro, ro)], agg_out.at[c, pl.ds(s * ro, ro)])

    pltpu.sync_copy(deg_sh.at[pl.ds(s * rz, rz)], deg_buf.at[pl.ds(0, rz)])

    @pl.when(c == 0)
    def _():
        pltpu.sync_copy(deg_buf.at[pl.ds(0, rz)], deg0_out.at[pl.ds(s * rz, rz)])

    @pl.when(c == 1)
    def _():
        pltpu.sync_copy(deg_buf.at[pl.ds(0, rz)], deg1_out.at[pl.ds(s * rz, rz)])


@functools.lru_cache(maxsize=None)
def _make_p1():
    mesh = plsc.VectorSubcoreMesh(
        core_axis_name="c", subcore_axis_name="s", num_cores=NC, num_subcores=NS)
    return pl.kernel(
        _p1_body,
        out_type=(jax.ShapeDtypeStruct((NC, _ACC_ROWS, D_HALF), jnp.float32),
                  jax.ShapeDtypeStruct((_ACC_ROWS,), jnp.float32),
                  jax.ShapeDtypeStruct((_ACC_ROWS,), jnp.float32)),
        mesh=mesh,
        scratch_types=[
            pltpu.VMEM_SHARED((_ACC_ROWS, D_HALF), jnp.float32),
            pltpu.VMEM_SHARED((_ACC_ROWS,), jnp.float32),
            pltpu.VMEM((_CH, 128), jnp.int32),
            pltpu.VMEM((_CH, 128), jnp.int32),
            pltpu.VMEM((128, D_HALF), jnp.float32),
            pltpu.VMEM((128, D_HALF), jnp.float32),
            pltpu.VMEM((128,), jnp.float32),
            pltpu.VMEM((_ACC_ROWS // NS,), jnp.float32),
            pltpu.SemaphoreType.DMA,
            pltpu.SemaphoreType.DMA,
            pltpu.SemaphoreType.DMA,
            pltpu.SemaphoreType.DMA,
        ])


def _p3_body(h, e0r, e1r, hi_out, hj_out, e0_v, e1_v, buf_i, buf_j, sem_i, sem_j):
    c = lax.axis_index("c")
    s = lax.axis_index("s")
    w = c * NS + s
    pltpu.sync_copy(e0r, e0_v)
    pltpu.sync_copy(e1r, e1_v)
    for b in range(_KB3):
        row = w * _KB3 + b
        cp_i = pltpu.async_copy(h.at[e0_v.at[row]], buf_i, sem_i)
        cp_j = pltpu.async_copy(h.at[e1_v.at[row]], buf_j, sem_j)
        cp_i.wait()
        cp_j.wait()
        base = w * _PPW + b * 128
        pltpu.sync_copy(buf_i, hi_out.at[pl.ds(base, 128)])
        pltpu.sync_copy(buf_j, hj_out.at[pl.ds(base, 128)])


@functools.lru_cache(maxsize=None)
def _make_p3():
    mesh = plsc.VectorSubcoreMesh(
        core_axis_name="c", subcore_axis_name="s", num_cores=NC, num_subcores=NS)
    return pl.kernel(
        _p3_body,
        out_type=(jax.ShapeDtypeStruct((_LPAD, D_FEAT), jnp.float32),
                  jax.ShapeDtypeStruct((_LPAD, D_FEAT), jnp.float32)),
        mesh=mesh,
        scratch_types=[
            pltpu.VMEM((_LPAD // 128, 128), jnp.int32),
            pltpu.VMEM((_LPAD // 128, 128), jnp.int32),
            pltpu.VMEM((128, D_FEAT), jnp.float32),
            pltpu.VMEM((128, D_FEAT), jnp.float32),
            pltpu.SemaphoreType.DMA,
            pltpu.SemaphoreType.DMA,
        ])


def _gnn_mm(a0_ref, a1_ref, d0_ref, d1_ref, w0_ref, w1_ref, b_ref, out_ref):
    r = 1.0 / jnp.maximum(d0_ref[...] + d1_ref[...], 1.0)
    acc = jnp.dot(a0_ref[0] * r, w0_ref[0], preferred_element_type=jnp.float32)
    acc += jnp.dot(a1_ref[0] * r, w1_ref[0], preferred_element_type=jnp.float32)
    out_ref[...] = jnp.maximum(acc + b_ref[...], 0.0)


def _mlp(hi_ref, hj_ref, w1_ref, b1_ref, w2_ref, b2_ref, out_ref):
    z = hi_ref[...] * hj_ref[...]
    t = jnp.dot(z, w1_ref[...], preferred_element_type=jnp.float32) + b1_ref[...]
    t = jnp.maximum(t, 0.0)
    logit = jnp.sum(t * w2_ref[...], axis=1, keepdims=True) + b2_ref[...]
    out_ref[...] = 1.0 / (1.0 + jnp.exp(-logit))


def kernel(x, edges, adj, W_gnn, b_gnn, W1, b1, W2, b2):
    x = x.astype(jnp.float32)
    src = adj[0].astype(jnp.int32)
    dst = adj[1].astype(jnp.int32)
    e0 = edges[0].astype(jnp.int32)
    e1 = edges[1].astype(jnp.int32)

    xi = x.reshape(N_NODES, 2, D_HALF).reshape(2 * N_NODES, D_HALF)
    pad = _EPAD - N_ADJ
    src_p = jnp.concatenate([src, jnp.zeros((pad,), jnp.int32)])
    dst_p = jnp.concatenate([dst, jnp.full((pad,), N_NODES, jnp.int32)])
    src_all = jnp.concatenate([2 * src_p, 2 * src_p + 1]).reshape(_ROWS1, 128)
    dst_all = jnp.concatenate([dst_p, dst_p]).reshape(_ROWS1, 128)
    z2 = jnp.zeros((_ACC_ROWS, D_HALF), jnp.float32)
    z1 = jnp.zeros((_ACC_ROWS,), jnp.float32)
    o1 = jnp.ones((128,), jnp.float32)

    agg, deg0, deg1 = _make_p1()(xi, src_all, dst_all, z2, z1, o1)
    d0 = deg0.reshape(_ACC_ROWS, 1)
    d1 = deg1.reshape(_ACC_ROWS, 1)

    bm = 1000
    h = pl.pallas_call(
        _gnn_mm,
        grid=(N_NODES // bm,),
        in_specs=[
            pl.BlockSpec((1, bm, D_HALF), lambda i: (0, i, 0)),
            pl.BlockSpec((1, bm, D_HALF), lambda i: (1, i, 0)),
            pl.BlockSpec((bm, 1), lambda i: (i, 0)),
            pl.BlockSpec((bm, 1), lambda i: (i, 0)),
            pl.BlockSpec((1, D_HALF, D_FEAT), lambda i: (0, 0, 0)),
            pl.BlockSpec((1, D_HALF, D_FEAT), lambda i: (1, 0, 0)),
            pl.BlockSpec((1, D_FEAT), lambda i: (0, 0)),
        ],
        out_specs=pl.BlockSpec((bm, D_FEAT), lambda i: (i, 0)),
        out_shape=jax.ShapeDtypeStruct((N_NODES, D_FEAT), jnp.float32),
    )(agg, agg, d0, d1, W_gnn.reshape(2, D_HALF, D_FEAT),
      W_gnn.reshape(2, D_HALF, D_FEAT), b_gnn.reshape(1, D_FEAT))

    lpad = _LPAD - N_LINK
    e0_p = jnp.concatenate([e0, jnp.zeros((lpad,), jnp.int32)]).reshape(_LPAD // 128, 128)
    e1_p = jnp.concatenate([e1, jnp.zeros((lpad,), jnp.int32)]).reshape(_LPAD // 128, 128)
    hi, hj = _make_p3()(h, e0_p, e1_p)

    bl = 1024
    logits = pl.pallas_call(
        _mlp,
        grid=(_LPAD // bl,),
        in_specs=[
            pl.BlockSpec((bl, D_FEAT), lambda i: (i, 0)),
            pl.BlockSpec((bl, D_FEAT), lambda i: (i, 0)),
            pl.BlockSpec((D_FEAT, D_HID), lambda i: (0, 0)),
            pl.BlockSpec((1, D_HID), lambda i: (0, 0)),
            pl.BlockSpec((1, D_HID), lambda i: (0, 0)),
            pl.BlockSpec((1, 1), lambda i: (0, 0)),
        ],
        out_specs=pl.BlockSpec((bl, 1), lambda i: (i, 0)),
        out_shape=jax.ShapeDtypeStruct((_LPAD, 1), jnp.float32),
    )(hi, hj, W1, b1.reshape(1, D_HID), W2.reshape(1, D_HID), b2.reshape(1, 1))

    return logits[:N_LINK, 0]


# speedup vs baseline: 1.0834x; 1.0700x over previous
import functools

import jax
import jax.numpy as jnp
from jax import lax
from jax.experimental import pallas as pl
from jax.experimental.pallas import tpu as pltpu
from jax.experimental.pallas import tpu_sc as plsc

N_NODES = 10000
D_FEAT = 256
D_HALF = 128
D_HID = 128
N_ADJ = 160000
N_LINK = 20000

NC = 2
NS = 16
NW = NC * NS

_KB1 = 80
_EPW = _KB1 * 128
_EPAD = NS * _EPW
_ROWS1 = 2 * _EPAD // 128
_ACC_ROWS = 10112
_CH = 40

_BLK3 = 64
_KB3 = 10
_PPW = _KB3 * _BLK3
_LPAD = NW * _PPW
_NBLK3 = _LPAD // _BLK3

def _p1_body(xi, srcr, dstr, z2, z1, o1, agg_out, deg0_out, deg1_out,
             acc_sh, deg_sh, src_v, dst_v, buf_a, buf_b, ones_v, deg_buf,
             sem_a, sem_b, sem_s, sem_d):
    c = lax.axis_index("c")
    s = lax.axis_index("s")
    w = c * NS + s

    pltpu.sync_copy(o1, ones_v)

    rz = _ACC_ROWS // NS
    pltpu.sync_copy(z2.at[pl.ds(s * rz, rz)], acc_sh.at[pl.ds(s * rz, rz)])
    pltpu.sync_copy(z1.at[pl.ds(s * rz, rz)], deg_buf.at[pl.ds(0, rz)])
    pltpu.sync_copy(deg_buf.at[pl.ds(0, rz)], deg_sh.at[pl.ds(s * rz, rz)])

    plsc.subcore_barrier()

    bufs = (buf_a, buf_b)
    sems = (sem_a, sem_b)

    def wait_gather(b_par):
        pltpu.make_async_copy(xi.at[src_v.at[0]], bufs[b_par], sems[b_par]).wait()

    def issue_scatter(b_par, b):
        pltpu.async_copy(bufs[b_par], acc_sh.at[dst_v.at[b]], sem_s, add=True)

    def wait_scatter(b_par):
        pltpu.make_async_copy(bufs[b_par], acc_sh.at[dst_v.at[0]], sem_s).wait()

    for hp in range(_KB1 // _CH):
        base = w * _KB1 + hp * _CH
        pltpu.sync_copy(srcr.at[pl.ds(base, _CH)], src_v)
        pltpu.sync_copy(dstr.at[pl.ds(base, _CH)], dst_v)
        deg_on = c == hp

        def deg_scatter(b):
            @pl.when(deg_on)
            def _():
                pltpu.async_copy(ones_v, deg_sh.at[dst_v.at[b]], sem_d,
                                 add=True)

        pltpu.async_copy(xi.at[src_v.at[0]], buf_a, sem_a)
        pltpu.async_copy(xi.at[src_v.at[1]], buf_b, sem_b)
        wait_gather(0)
        issue_scatter(0, 0)
        deg_scatter(0)

        def halfpass(i, carry):
            b1 = 2 * i + 1
            wait_scatter(0)
            pltpu.async_copy(xi.at[src_v.at[b1 + 1]], buf_a, sem_a)
            wait_gather(1)
            issue_scatter(1, b1)
            deg_scatter(b1)

            b2 = b1 + 1
            wait_scatter(1)
            pltpu.async_copy(xi.at[src_v.at[b2 + 1]], buf_b, sem_b)
            wait_gather(0)
            issue_scatter(0, b2)
            deg_scatter(b2)
            return carry

        lax.fori_loop(0, _CH // 2 - 1, halfpass, 0)

        wait_scatter(0)
        wait_gather(1)
        issue_scatter(1, _CH - 1)
        deg_scatter(_CH - 1)
        wait_scatter(1)

        @pl.when(deg_on)
        def _():
            for _i in range(_CH):
                pltpu.make_async_copy(ones_v, deg_sh.at[dst_v.at[0]], sem_d).wait()

    plsc.subcore_barrier()

    ro = _ACC_ROWS // NS
    pltpu.sync_copy(acc_sh.at[pl.ds(s * ro, ro)], agg_out.at[c, pl.ds(s * ro, ro)])

    pltpu.sync_copy(deg_sh.at[pl.ds(s * rz, rz)], deg_buf.at[pl.ds(0, rz)])

    @pl.when(c == 0)
    def _():
        pltpu.sync_copy(deg_buf.at[pl.ds(0, rz)], deg0_out.at[pl.ds(s * rz, rz)])

    @pl.when(c == 1)
    def _():
        pltpu.sync_copy(deg_buf.at[pl.ds(0, rz)], deg1_out.at[pl.ds(s * rz, rz)])


@functools.lru_cache(maxsize=None)
def _make_p1():
    mesh = plsc.VectorSubcoreMesh(
        core_axis_name="c", subcore_axis_name="s", num_cores=NC, num_subcores=NS)
    return pl.kernel(
        _p1_body,
        out_type=(jax.ShapeDtypeStruct((NC, _ACC_ROWS, D_HALF), jnp.float32),
                  jax.ShapeDtypeStruct((_ACC_ROWS,), jnp.float32),
                  jax.ShapeDtypeStruct((_ACC_ROWS,), jnp.float32)),
        mesh=mesh,
        scratch_types=[
            pltpu.VMEM_SHARED((_ACC_ROWS, D_HALF), jnp.float32),
            pltpu.VMEM_SHARED((_ACC_ROWS,), jnp.float32),
            pltpu.VMEM((_CH, 128), jnp.int32),
            pltpu.VMEM((_CH, 128), jnp.int32),
            pltpu.VMEM((128, D_HALF), jnp.float32),
            pltpu.VMEM((128, D_HALF), jnp.float32),
            pltpu.VMEM((128,), jnp.float32),
            pltpu.VMEM((_ACC_ROWS // NS,), jnp.float32),
            pltpu.SemaphoreType.DMA,
            pltpu.SemaphoreType.DMA,
            pltpu.SemaphoreType.DMA,
            pltpu.SemaphoreType.DMA,
        ])


def _p3_body(h, epr, hi_out, hj_out, eb_v, b0, b1, b2,
             g0, g1, g2, w0, w1, w2):
    c = lax.axis_index("c")
    s = lax.axis_index("s")
    w = c * NS + s
    row0 = w * _KB3
    base8 = (row0 // 8) * 8
    off = row0 - base8
    pltpu.sync_copy(epr.at[pl.ds(base8, 16)], eb_v)

    bufs = (b0, b1, b2)
    gs = (g0, g1, g2)
    ws = (w0, w1, w2)

    def ig(p, k):
        pltpu.async_copy(h.at[eb_v.at[off + k]], bufs[p], gs[p])

    def wg(p):
        pltpu.make_async_copy(h.at[eb_v.at[0]], bufs[p], gs[p]).wait()

    def iw(p, k):
        base = w * _PPW + k * _BLK3
        pltpu.async_copy(bufs[p].at[pl.ds(0, 64)],
                         hi_out.at[pl.ds(base, 64)], ws[p])
        pltpu.async_copy(bufs[p].at[pl.ds(64, 64)],
                         hj_out.at[pl.ds(base, 64)], ws[p])

    def ww(p):
        pltpu.make_async_copy(bufs[p].at[pl.ds(0, 64)],
                              hi_out.at[pl.ds(0, 64)], ws[p]).wait()
        pltpu.make_async_copy(bufs[p].at[pl.ds(64, 64)],
                              hj_out.at[pl.ds(0, 64)], ws[p]).wait()

    ig(0, 0)
    ig(1, 1)
    ig(2, 2)
    for k in range(_KB3):
        p = k % 3
        wg(p)
        iw(p, k)
        if k + 3 < _KB3:
            ww(p)
            ig(p, k + 3)
    ww((_KB3 - 3) % 3)
    ww((_KB3 - 2) % 3)
    ww((_KB3 - 1) % 3)


@functools.lru_cache(maxsize=None)
def _make_p3():
    mesh = plsc.VectorSubcoreMesh(
        core_axis_name="c", subcore_axis_name="s", num_cores=NC, num_subcores=NS)
    return pl.kernel(
        _p3_body,
        out_type=(jax.ShapeDtypeStruct((_LPAD, D_FEAT), jnp.float32),
                  jax.ShapeDtypeStruct((_LPAD, D_FEAT), jnp.float32)),
        mesh=mesh,
        scratch_types=[
            pltpu.VMEM((16, 128), jnp.int32),
            pltpu.VMEM((128, D_FEAT), jnp.float32),
            pltpu.VMEM((128, D_FEAT), jnp.float32),
            pltpu.VMEM((128, D_FEAT), jnp.float32),
            pltpu.SemaphoreType.DMA,
            pltpu.SemaphoreType.DMA,
            pltpu.SemaphoreType.DMA,
            pltpu.SemaphoreType.DMA,
            pltpu.SemaphoreType.DMA,
            pltpu.SemaphoreType.DMA,
        ])


def _gnn_mm(a0_ref, a1_ref, d0_ref, d1_ref, w0_ref, w1_ref, b_ref, out_ref):
    r = 1.0 / jnp.maximum(d0_ref[...] + d1_ref[...], 1.0)
    acc = jnp.dot(a0_ref[0] * r, w0_ref[0], preferred_element_type=jnp.float32)
    acc += jnp.dot(a1_ref[0] * r, w1_ref[0], preferred_element_type=jnp.float32)
    out_ref[...] = jnp.maximum(acc + b_ref[...], 0.0)


def _mlp(hi_ref, hj_ref, w1_ref, b1_ref, w2_ref, b2_ref, out_ref):
    z = hi_ref[...] * hj_ref[...]
    t = jnp.dot(z, w1_ref[...], preferred_element_type=jnp.float32) + b1_ref[...]
    t = jnp.maximum(t, 0.0)
    logit = jnp.sum(t * w2_ref[...], axis=1, keepdims=True) + b2_ref[...]
    out_ref[...] = 1.0 / (1.0 + jnp.exp(-logit))


def kernel(x, edges, adj, W_gnn, b_gnn, W1, b1, W2, b2):
    x = x.astype(jnp.float32)
    src = adj[0].astype(jnp.int32)
    dst = adj[1].astype(jnp.int32)
    e0 = edges[0].astype(jnp.int32)
    e1 = edges[1].astype(jnp.int32)

    xi = x.reshape(N_NODES, 2, D_HALF).reshape(2 * N_NODES, D_HALF)
    pad = _EPAD - N_ADJ
    src_p = jnp.concatenate([src, jnp.zeros((pad,), jnp.int32)])
    dst_p = jnp.concatenate([dst, jnp.full((pad,), N_NODES, jnp.int32)])
    src_all = jnp.concatenate([2 * src_p, 2 * src_p + 1]).reshape(_ROWS1, 128)
    dst_all = jnp.concatenate([dst_p, dst_p]).reshape(_ROWS1, 128)
    z2 = jnp.zeros((_ACC_ROWS, D_HALF), jnp.float32)
    z1 = jnp.zeros((_ACC_ROWS,), jnp.float32)
    o1 = jnp.ones((128,), jnp.float32)

    agg, deg0, deg1 = _make_p1()(xi, src_all, dst_all, z2, z1, o1)
    d0 = deg0.reshape(_ACC_ROWS, 1)
    d1 = deg1.reshape(_ACC_ROWS, 1)

    bm = 1000
    h = pl.pallas_call(
        _gnn_mm,
        grid=(N_NODES // bm,),
        in_specs=[
            pl.BlockSpec((1, bm, D_HALF), lambda i: (0, i, 0)),
            pl.BlockSpec((1, bm, D_HALF), lambda i: (1, i, 0)),
            pl.BlockSpec((bm, 1), lambda i: (i, 0)),
            pl.BlockSpec((bm, 1), lambda i: (i, 0)),
            pl.BlockSpec((1, D_HALF, D_FEAT), lambda i: (0, 0, 0)),
            pl.BlockSpec((1, D_HALF, D_FEAT), lambda i: (1, 0, 0)),
            pl.BlockSpec((1, D_FEAT), lambda i: (0, 0)),
        ],
        out_specs=pl.BlockSpec((bm, D_FEAT), lambda i: (i, 0)),
        out_shape=jax.ShapeDtypeStruct((N_NODES, D_FEAT), jnp.float32),
    )(agg, agg, d0, d1, W_gnn.reshape(2, D_HALF, D_FEAT),
      W_gnn.reshape(2, D_HALF, D_FEAT), b_gnn.reshape(1, D_FEAT))

    lpad = _LPAD - N_LINK
    e0_p = jnp.concatenate([e0, jnp.zeros((lpad,), jnp.int32)]).reshape(_NBLK3, _BLK3)
    e1_p = jnp.concatenate([e1, jnp.zeros((lpad,), jnp.int32)]).reshape(_NBLK3, _BLK3)
    epair = jnp.concatenate([e0_p, e1_p], axis=1)
    hi, hj = _make_p3()(h, epair)

    bl = 1024
    logits = pl.pallas_call(
        _mlp,
        grid=(_LPAD // bl,),
        in_specs=[
            pl.BlockSpec((bl, D_FEAT), lambda i: (i, 0)),
            pl.BlockSpec((bl, D_FEAT), lambda i: (i, 0)),
            pl.BlockSpec((D_FEAT, D_HID), lambda i: (0, 0)),
            pl.BlockSpec((1, D_HID), lambda i: (0, 0)),
            pl.BlockSpec((1, D_HID), lambda i: (0, 0)),
            pl.BlockSpec((1, 1), lambda i: (0, 0)),
        ],
        out_specs=pl.BlockSpec((bl, 1), lambda i: (i, 0)),
        out_shape=jax.ShapeDtypeStruct((_LPAD, 1), jnp.float32),
    )(hi, hj, W1, b1.reshape(1, D_HID), W2.reshape(1, D_HID), b2.reshape(1, 1))

    return logits[:N_LINK, 0]


# speedup vs baseline: 1.0864x; 1.0028x over previous
import functools

import jax
import jax.numpy as jnp
from jax import lax
from jax.experimental import pallas as pl
from jax.experimental.pallas import tpu as pltpu
from jax.experimental.pallas import tpu_sc as plsc

N_NODES = 10000
D_FEAT = 256
D_HALF = 128
D_HID = 128
N_ADJ = 160000
N_LINK = 20000

NC = 2
NS = 16
NW = NC * NS

_KB1 = 80
_EPW = _KB1 * 128
_EPAD = NS * _EPW
_ROWS1 = 2 * _EPAD // 128
_ACC_ROWS = 10112
_CH = 40

_BLK3 = 64
_KB3 = 10
_PPW = _KB3 * _BLK3
_LPAD = NW * _PPW
_NBLK3 = _LPAD // _BLK3

def _p1_body(xi, srcr, dstr, z2, z1, o1, agg_out, deg0_out, deg1_out,
             acc_sh, deg_sh, src_v, dst_v, buf_a, buf_b, ones_v, deg_buf,
             sem_a, sem_b, sem_s, sem_d):
    c = lax.axis_index("c")
    s = lax.axis_index("s")
    w = c * NS + s

    pltpu.sync_copy(o1, ones_v)

    rz = _ACC_ROWS // NS
    pltpu.sync_copy(z2, buf_a)
    for k in range(4):
        pltpu.sync_copy(buf_a, acc_sh.at[pl.ds(s * rz + k * 128, 128)])
    pltpu.sync_copy(buf_a.at[pl.ds(0, rz - 512)],
                    acc_sh.at[pl.ds(s * rz + 512, rz - 512)])
    pltpu.sync_copy(z1, deg_buf.at[pl.ds(0, rz)])
    pltpu.sync_copy(deg_buf.at[pl.ds(0, rz)], deg_sh.at[pl.ds(s * rz, rz)])

    plsc.subcore_barrier()

    bufs = (buf_a, buf_b)
    sems = (sem_a, sem_b)

    def wait_gather(b_par):
        pltpu.make_async_copy(xi.at[src_v.at[0]], bufs[b_par], sems[b_par]).wait()

    def issue_scatter(b_par, b):
        pltpu.async_copy(bufs[b_par], acc_sh.at[dst_v.at[b]], sem_s, add=True)

    def wait_scatter(b_par):
        pltpu.make_async_copy(bufs[b_par], acc_sh.at[dst_v.at[0]], sem_s).wait()

    for hp in range(_KB1 // _CH):
        base = w * _KB1 + hp * _CH
        pltpu.sync_copy(srcr.at[pl.ds(base, _CH)], src_v)
        pltpu.sync_copy(dstr.at[pl.ds(base, _CH)], dst_v)
        deg_on = c == hp

        def deg_scatter(b):
            @pl.when(deg_on)
            def _():
                pltpu.async_copy(ones_v, deg_sh.at[dst_v.at[b]], sem_d,
                                 add=True)

        pltpu.async_copy(xi.at[src_v.at[0]], buf_a, sem_a)
        pltpu.async_copy(xi.at[src_v.at[1]], buf_b, sem_b)
        wait_gather(0)
        issue_scatter(0, 0)
        deg_scatter(0)

        def halfpass(i, carry):
            b1 = 2 * i + 1
            wait_scatter(0)
            pltpu.async_copy(xi.at[src_v.at[b1 + 1]], buf_a, sem_a)
            wait_gather(1)
            issue_scatter(1, b1)
            deg_scatter(b1)

            b2 = b1 + 1
            wait_scatter(1)
            pltpu.async_copy(xi.at[src_v.at[b2 + 1]], buf_b, sem_b)
            wait_gather(0)
            issue_scatter(0, b2)
            deg_scatter(b2)
            return carry

        lax.fori_loop(0, _CH // 2 - 1, halfpass, 0)

        wait_scatter(0)
        wait_gather(1)
        issue_scatter(1, _CH - 1)
        deg_scatter(_CH - 1)
        wait_scatter(1)

        @pl.when(deg_on)
        def _():
            for _i in range(_CH):
                pltpu.make_async_copy(ones_v, deg_sh.at[dst_v.at[0]], sem_d).wait()

    plsc.subcore_barrier()

    ro = _ACC_ROWS // NS
    pltpu.sync_copy(acc_sh.at[pl.ds(s * ro, ro)], agg_out.at[c, pl.ds(s * ro, ro)])

    pltpu.sync_copy(deg_sh.at[pl.ds(s * rz, rz)], deg_buf.at[pl.ds(0, rz)])

    @pl.when(c == 0)
    def _():
        pltpu.sync_copy(deg_buf.at[pl.ds(0, rz)], deg0_out.at[pl.ds(s * rz, rz)])

    @pl.when(c == 1)
    def _():
        pltpu.sync_copy(deg_buf.at[pl.ds(0, rz)], deg1_out.at[pl.ds(s * rz, rz)])


@functools.lru_cache(maxsize=None)
def _make_p1():
    mesh = plsc.VectorSubcoreMesh(
        core_axis_name="c", subcore_axis_name="s", num_cores=NC, num_subcores=NS)
    return pl.kernel(
        _p1_body,
        out_type=(jax.ShapeDtypeStruct((NC, _ACC_ROWS, D_HALF), jnp.float32),
                  jax.ShapeDtypeStruct((_ACC_ROWS,), jnp.float32),
                  jax.ShapeDtypeStruct((_ACC_ROWS,), jnp.float32)),
        mesh=mesh,
        scratch_types=[
            pltpu.VMEM_SHARED((_ACC_ROWS, D_HALF), jnp.float32),
            pltpu.VMEM_SHARED((_ACC_ROWS,), jnp.float32),
            pltpu.VMEM((_CH, 128), jnp.int32),
            pltpu.VMEM((_CH, 128), jnp.int32),
            pltpu.VMEM((128, D_HALF), jnp.float32),
            pltpu.VMEM((128, D_HALF), jnp.float32),
            pltpu.VMEM((128,), jnp.float32),
            pltpu.VMEM((_ACC_ROWS // NS,), jnp.float32),
            pltpu.SemaphoreType.DMA,
            pltpu.SemaphoreType.DMA,
            pltpu.SemaphoreType.DMA,
            pltpu.SemaphoreType.DMA,
        ])


def _p3_body(h, epr, hi_out, hj_out, eb_v, b0, b1, b2,
             g0, g1, g2, w0, w1, w2):
    c = lax.axis_index("c")
    s = lax.axis_index("s")
    w = c * NS + s
    row0 = w * _KB3
    base8 = (row0 // 8) * 8
    off = row0 - base8
    pltpu.sync_copy(epr.at[pl.ds(base8, 16)], eb_v)

    bufs = (b0, b1, b2)
    gs = (g0, g1, g2)
    ws = (w0, w1, w2)

    def ig(p, k):
        pltpu.async_copy(h.at[eb_v.at[off + k]], bufs[p], gs[p])

    def wg(p):
        pltpu.make_async_copy(h.at[eb_v.at[0]], bufs[p], gs[p]).wait()

    def iw(p, k):
        base = w * _PPW + k * _BLK3
        pltpu.async_copy(bufs[p].at[pl.ds(0, 64)],
                         hi_out.at[pl.ds(base, 64)], ws[p])
        pltpu.async_copy(bufs[p].at[pl.ds(64, 64)],
                         hj_out.at[pl.ds(base, 64)], ws[p])

    def ww(p):
        pltpu.make_async_copy(bufs[p].at[pl.ds(0, 64)],
                              hi_out.at[pl.ds(0, 64)], ws[p]).wait()
        pltpu.make_async_copy(bufs[p].at[pl.ds(64, 64)],
                              hj_out.at[pl.ds(0, 64)], ws[p]).wait()

    ig(0, 0)
    ig(1, 1)
    ig(2, 2)
    for k in range(_KB3):
        p = k % 3
        wg(p)
        iw(p, k)
        if k + 3 < _KB3:
            ww(p)
            ig(p, k + 3)
    ww((_KB3 - 3) % 3)
    ww((_KB3 - 2) % 3)
    ww((_KB3 - 1) % 3)


@functools.lru_cache(maxsize=None)
def _make_p3():
    mesh = plsc.VectorSubcoreMesh(
        core_axis_name="c", subcore_axis_name="s", num_cores=NC, num_subcores=NS)
    return pl.kernel(
        _p3_body,
        out_type=(jax.ShapeDtypeStruct((_LPAD, D_FEAT), jnp.float32),
                  jax.ShapeDtypeStruct((_LPAD, D_FEAT), jnp.float32)),
        mesh=mesh,
        scratch_types=[
            pltpu.VMEM((16, 128), jnp.int32),
            pltpu.VMEM((128, D_FEAT), jnp.float32),
            pltpu.VMEM((128, D_FEAT), jnp.float32),
            pltpu.VMEM((128, D_FEAT), jnp.float32),
            pltpu.SemaphoreType.DMA,
            pltpu.SemaphoreType.DMA,
            pltpu.SemaphoreType.DMA,
            pltpu.SemaphoreType.DMA,
            pltpu.SemaphoreType.DMA,
            pltpu.SemaphoreType.DMA,
        ])


def _gnn_mm(a0_ref, a1_ref, d0_ref, d1_ref, w0_ref, w1_ref, b_ref, out_ref):
    r = 1.0 / jnp.maximum(d0_ref[...] + d1_ref[...], 1.0)
    acc = jnp.dot(a0_ref[0] * r, w0_ref[0], preferred_element_type=jnp.float32)
    acc += jnp.dot(a1_ref[0] * r, w1_ref[0], preferred_element_type=jnp.float32)
    out_ref[...] = jnp.maximum(acc + b_ref[...], 0.0)


def _mlp(hi_ref, hj_ref, w1_ref, b1_ref, w2_ref, b2_ref, out_ref):
    z = hi_ref[...] * hj_ref[...]
    t = jnp.dot(z, w1_ref[...], preferred_element_type=jnp.float32) + b1_ref[...]
    t = jnp.maximum(t, 0.0)
    logit = jnp.sum(t * w2_ref[...], axis=1, keepdims=True) + b2_ref[...]
    out_ref[...] = 1.0 / (1.0 + jnp.exp(-logit))


def kernel(x, edges, adj, W_gnn, b_gnn, W1, b1, W2, b2):
    x = x.astype(jnp.float32)
    src = adj[0].astype(jnp.int32)
    dst = adj[1].astype(jnp.int32)
    e0 = edges[0].astype(jnp.int32)
    e1 = edges[1].astype(jnp.int32)

    xi = x.reshape(N_NODES, 2, D_HALF).reshape(2 * N_NODES, D_HALF)
    pad = _EPAD - N_ADJ
    src_p = jnp.concatenate([src, jnp.zeros((pad,), jnp.int32)])
    dst_p = jnp.concatenate([dst, jnp.full((pad,), N_NODES, jnp.int32)])
    src_all = jnp.concatenate([2 * src_p, 2 * src_p + 1]).reshape(_ROWS1, 128)
    dst_all = jnp.concatenate([dst_p, dst_p]).reshape(_ROWS1, 128)
    z2 = jnp.zeros((128, D_HALF), jnp.float32)
    z1 = jnp.zeros((_ACC_ROWS // NS,), jnp.float32)
    o1 = jnp.ones((128,), jnp.float32)

    agg, deg0, deg1 = _make_p1()(xi, src_all, dst_all, z2, z1, o1)
    d0 = deg0.reshape(_ACC_ROWS, 1)
    d1 = deg1.reshape(_ACC_ROWS, 1)

    bm = 1000
    h = pl.pallas_call(
        _gnn_mm,
        grid=(N_NODES // bm,),
        in_specs=[
            pl.BlockSpec((1, bm, D_HALF), lambda i: (0, i, 0)),
            pl.BlockSpec((1, bm, D_HALF), lambda i: (1, i, 0)),
            pl.BlockSpec((bm, 1), lambda i: (i, 0)),
            pl.BlockSpec((bm, 1), lambda i: (i, 0)),
            pl.BlockSpec((1, D_HALF, D_FEAT), lambda i: (0, 0, 0)),
            pl.BlockSpec((1, D_HALF, D_FEAT), lambda i: (1, 0, 0)),
            pl.BlockSpec((1, D_FEAT), lambda i: (0, 0)),
        ],
        out_specs=pl.BlockSpec((bm, D_FEAT), lambda i: (i, 0)),
        out_shape=jax.ShapeDtypeStruct((N_NODES, D_FEAT), jnp.float32),
    )(agg, agg, d0, d1, W_gnn.reshape(2, D_HALF, D_FEAT),
      W_gnn.reshape(2, D_HALF, D_FEAT), b_gnn.reshape(1, D_FEAT))

    lpad = _LPAD - N_LINK
    e0_p = jnp.concatenate([e0, jnp.zeros((lpad,), jnp.int32)]).reshape(_NBLK3, _BLK3)
    e1_p = jnp.concatenate([e1, jnp.zeros((lpad,), jnp.int32)]).reshape(_NBLK3, _BLK3)
    epair = jnp.concatenate([e0_p, e1_p], axis=1)
    hi, hj = _make_p3()(h, epair)

    bl = 1024
    logits = pl.pallas_call(
        _mlp,
        grid=(_LPAD // bl,),
        in_specs=[
            pl.BlockSpec((bl, D_FEAT), lambda i: (i, 0)),
            pl.BlockSpec((bl, D_FEAT), lambda i: (i, 0)),
            pl.BlockSpec((D_FEAT, D_HID), lambda i: (0, 0)),
            pl.BlockSpec((1, D_HID), lambda i: (0, 0)),
            pl.BlockSpec((1, D_HID), lambda i: (0, 0)),
            pl.BlockSpec((1, 1), lambda i: (0, 0)),
        ],
        out_specs=pl.BlockSpec((bl, 1), lambda i: (i, 0)),
        out_shape=jax.ShapeDtypeStruct((_LPAD, 1), jnp.float32),
    )(hi, hj, W1, b1.reshape(1, D_HID), W2.reshape(1, D_HID), b2.reshape(1, 1))

    return logits[:N_LINK, 0]


# speedup vs baseline: 1.1005x; 1.0130x over previous
import functools

import jax
import jax.numpy as jnp
from jax import lax
from jax.experimental import pallas as pl
from jax.experimental.pallas import tpu as pltpu
from jax.experimental.pallas import tpu_sc as plsc

N_NODES = 10000
D_FEAT = 256
D_HALF = 128
D_HID = 128
N_ADJ = 160000
N_LINK = 20000

NC = 2
NS = 16
NW = NC * NS

_KB1 = 80
_EPW = _KB1 * 128
_EPAD = NS * _EPW
_ROWS1 = 2 * _EPAD // 128
_ACC_ROWS = 10112
_CH = 40

_BLK3 = 64
_KB3 = 10
_PPW = _KB3 * _BLK3
_LPAD = NW * _PPW
_NBLK3 = _LPAD // _BLK3

def _p1_body(xi, srcr, dstr, z2, z1, o1, agg_out, deg0_out, deg1_out,
             acc_sh, deg_sh, src_v, dst_v, buf_a, buf_b, ones_v, deg_buf,
             sem_a, sem_b, sem_s, sem_d):
    c = lax.axis_index("c")
    s = lax.axis_index("s")
    w = c * NS + s

    pltpu.sync_copy(o1, ones_v)

    rz = _ACC_ROWS // NS
    pltpu.sync_copy(z2, buf_a)
    for k in range(4):
        pltpu.sync_copy(buf_a, acc_sh.at[pl.ds(s * rz + k * 128, 128)])
    pltpu.sync_copy(buf_a.at[pl.ds(0, rz - 512)],
                    acc_sh.at[pl.ds(s * rz + 512, rz - 512)])
    pltpu.sync_copy(z1, deg_buf.at[pl.ds(0, rz)])
    pltpu.sync_copy(deg_buf.at[pl.ds(0, rz)], deg_sh.at[pl.ds(s * rz, rz)])

    plsc.subcore_barrier()

    bufs = (buf_a, buf_b)
    sems = (sem_a, sem_b)

    def wait_gather(b_par):
        pltpu.make_async_copy(xi.at[src_v.at[0]], bufs[b_par], sems[b_par]).wait()

    def issue_scatter(b_par, b):
        pltpu.async_copy(bufs[b_par], acc_sh.at[dst_v.at[b]], sem_s, add=True)

    def wait_scatter(b_par):
        pltpu.make_async_copy(bufs[b_par], acc_sh.at[dst_v.at[0]], sem_s).wait()

    for hp in range(_KB1 // _CH):
        base = w * _KB1 + hp * _CH
        pltpu.sync_copy(srcr.at[pl.ds(base, _CH)], src_v)
        pltpu.sync_copy(dstr.at[pl.ds(base, _CH)], dst_v)
        deg_on = c == hp

        def deg_scatter(b):
            @pl.when(deg_on)
            def _():
                pltpu.async_copy(ones_v, deg_sh.at[dst_v.at[b]], sem_d,
                                 add=True)

        pltpu.async_copy(xi.at[src_v.at[0]], buf_a, sem_a)
        pltpu.async_copy(xi.at[src_v.at[1]], buf_b, sem_b)
        wait_gather(0)
        issue_scatter(0, 0)
        deg_scatter(0)

        def halfpass(i, carry):
            b1 = 2 * i + 1
            wait_scatter(0)
            pltpu.async_copy(xi.at[src_v.at[b1 + 1]], buf_a, sem_a)
            wait_gather(1)
            issue_scatter(1, b1)
            deg_scatter(b1)

            b2 = b1 + 1
            wait_scatter(1)
            pltpu.async_copy(xi.at[src_v.at[b2 + 1]], buf_b, sem_b)
            wait_gather(0)
            issue_scatter(0, b2)
            deg_scatter(b2)
            return carry

        lax.fori_loop(0, _CH // 2 - 1, halfpass, 0)

        wait_scatter(0)
        wait_gather(1)
        issue_scatter(1, _CH - 1)
        deg_scatter(_CH - 1)
        wait_scatter(1)

        @pl.when(deg_on)
        def _():
            for _i in range(_CH):
                pltpu.make_async_copy(ones_v, deg_sh.at[dst_v.at[0]], sem_d).wait()

    plsc.subcore_barrier()

    ro = _ACC_ROWS // NS
    pltpu.sync_copy(acc_sh.at[pl.ds(s * ro, ro)], agg_out.at[c, pl.ds(s * ro, ro)])

    pltpu.sync_copy(deg_sh.at[pl.ds(s * rz, rz)], deg_buf.at[pl.ds(0, rz)])

    @pl.when(c == 0)
    def _():
        pltpu.sync_copy(deg_buf.at[pl.ds(0, rz)], deg0_out.at[pl.ds(s * rz, rz)])

    @pl.when(c == 1)
    def _():
        pltpu.sync_copy(deg_buf.at[pl.ds(0, rz)], deg1_out.at[pl.ds(s * rz, rz)])


@functools.lru_cache(maxsize=None)
def _make_p1():
    mesh = plsc.VectorSubcoreMesh(
        core_axis_name="c", subcore_axis_name="s", num_cores=NC, num_subcores=NS)
    return pl.kernel(
        _p1_body,
        out_type=(jax.ShapeDtypeStruct((NC, _ACC_ROWS, D_HALF), jnp.float32),
                  jax.ShapeDtypeStruct((_ACC_ROWS,), jnp.float32),
                  jax.ShapeDtypeStruct((_ACC_ROWS,), jnp.float32)),
        mesh=mesh,
        scratch_types=[
            pltpu.VMEM_SHARED((_ACC_ROWS, D_HALF), jnp.float32),
            pltpu.VMEM_SHARED((_ACC_ROWS,), jnp.float32),
            pltpu.VMEM((_CH, 128), jnp.int32),
            pltpu.VMEM((_CH, 128), jnp.int32),
            pltpu.VMEM((128, D_HALF), jnp.float32),
            pltpu.VMEM((128, D_HALF), jnp.float32),
            pltpu.VMEM((128,), jnp.float32),
            pltpu.VMEM((_ACC_ROWS // NS,), jnp.float32),
            pltpu.SemaphoreType.DMA,
            pltpu.SemaphoreType.DMA,
            pltpu.SemaphoreType.DMA,
            pltpu.SemaphoreType.DMA,
        ])


def _p3_body(h, epr, hi_out, hj_out, eb_v, b0, b1, b2,
             g0, g1, g2, w0, w1, w2):
    c = lax.axis_index("c")
    s = lax.axis_index("s")
    w = c * NS + s
    row0 = w * _KB3
    base8 = (row0 // 8) * 8
    off = row0 - base8
    pltpu.sync_copy(epr.at[pl.ds(base8, 16)], eb_v)

    bufs = (b0, b1, b2)
    gs = (g0, g1, g2)
    ws = (w0, w1, w2)

    def ig(p, k):
        pltpu.async_copy(h.at[eb_v.at[off + k]], bufs[p], gs[p])

    def wg(p):
        pltpu.make_async_copy(h.at[eb_v.at[0]], bufs[p], gs[p]).wait()

    def iw(p, k):
        base = w * _PPW + k * _BLK3
        pltpu.async_copy(bufs[p].at[pl.ds(0, 64)],
                         hi_out.at[pl.ds(base, 64)], ws[p])
        pltpu.async_copy(bufs[p].at[pl.ds(64, 64)],
                         hj_out.at[pl.ds(base, 64)], ws[p])

    def ww(p):
        pltpu.make_async_copy(bufs[p].at[pl.ds(0, 64)],
                              hi_out.at[pl.ds(0, 64)], ws[p]).wait()
        pltpu.make_async_copy(bufs[p].at[pl.ds(64, 64)],
                              hj_out.at[pl.ds(0, 64)], ws[p]).wait()

    ig(0, 0)
    ig(1, 1)
    ig(2, 2)
    for k in range(_KB3):
        p = k % 3
        wg(p)
        iw(p, k)
        if k + 3 < _KB3:
            ww(p)
            ig(p, k + 3)
    ww((_KB3 - 3) % 3)
    ww((_KB3 - 2) % 3)
    ww((_KB3 - 1) % 3)


@functools.lru_cache(maxsize=None)
def _make_p3():
    mesh = plsc.VectorSubcoreMesh(
        core_axis_name="c", subcore_axis_name="s", num_cores=NC, num_subcores=NS)
    return pl.kernel(
        _p3_body,
        out_type=(jax.ShapeDtypeStruct((_LPAD, D_FEAT), jnp.float32),
                  jax.ShapeDtypeStruct((_LPAD, D_FEAT), jnp.float32)),
        mesh=mesh,
        scratch_types=[
            pltpu.VMEM((16, 128), jnp.int32),
            pltpu.VMEM((128, D_FEAT), jnp.float32),
            pltpu.VMEM((128, D_FEAT), jnp.float32),
            pltpu.VMEM((128, D_FEAT), jnp.float32),
            pltpu.SemaphoreType.DMA,
            pltpu.SemaphoreType.DMA,
            pltpu.SemaphoreType.DMA,
            pltpu.SemaphoreType.DMA,
            pltpu.SemaphoreType.DMA,
            pltpu.SemaphoreType.DMA,
        ])


def _gnn_mm(a0_ref, a1_ref, d0_ref, d1_ref, w0_ref, w1_ref, b_ref, out_ref):
    r = 1.0 / jnp.maximum(d0_ref[...] + d1_ref[...], 1.0)
    acc = jnp.dot(a0_ref[0] * r, w0_ref[0], preferred_element_type=jnp.float32)
    acc += jnp.dot(a1_ref[0] * r, w1_ref[0], preferred_element_type=jnp.float32)
    out_ref[...] = jnp.maximum(acc + b_ref[...], 0.0)


def _mlp(hi_ref, hj_ref, w1_ref, b1_ref, w2_ref, b2_ref, out_ref):
    z = hi_ref[...] * hj_ref[...]
    t = jnp.dot(z, w1_ref[...], preferred_element_type=jnp.float32) + b1_ref[...]
    t = jnp.maximum(t, 0.0)
    logit = jnp.sum(t * w2_ref[...], axis=1, keepdims=True) + b2_ref[...]
    out_ref[...] = 1.0 / (1.0 + jnp.exp(-logit))


def kernel(x, edges, adj, W_gnn, b_gnn, W1, b1, W2, b2):
    x = x.astype(jnp.float32)
    src = adj[0].astype(jnp.int32)
    dst = adj[1].astype(jnp.int32)
    e0 = edges[0].astype(jnp.int32)
    e1 = edges[1].astype(jnp.int32)

    xi = x.reshape(N_NODES, 2, D_HALF).reshape(2 * N_NODES, D_HALF)
    pad = _EPAD - N_ADJ
    src_p = jnp.concatenate([src, jnp.zeros((pad,), jnp.int32)])
    dst_p = jnp.concatenate([dst, jnp.full((pad,), N_NODES, jnp.int32)])
    src_all = jnp.concatenate([2 * src_p, 2 * src_p + 1]).reshape(_ROWS1, 128)
    dst_all = jnp.concatenate([dst_p, dst_p]).reshape(_ROWS1, 128)
    z2 = jnp.zeros((128, D_HALF), jnp.float32)
    z1 = jnp.zeros((_ACC_ROWS // NS,), jnp.float32)
    o1 = jnp.ones((128,), jnp.float32)

    agg, deg0, deg1 = _make_p1()(xi, src_all, dst_all, z2, z1, o1)
    d0 = deg0.reshape(_ACC_ROWS, 1)
    d1 = deg1.reshape(_ACC_ROWS, 1)

    bm = 2000
    h = pl.pallas_call(
        _gnn_mm,
        grid=(N_NODES // bm,),
        in_specs=[
            pl.BlockSpec((1, bm, D_HALF), lambda i: (0, i, 0)),
            pl.BlockSpec((1, bm, D_HALF), lambda i: (1, i, 0)),
            pl.BlockSpec((bm, 1), lambda i: (i, 0)),
            pl.BlockSpec((bm, 1), lambda i: (i, 0)),
            pl.BlockSpec((1, D_HALF, D_FEAT), lambda i: (0, 0, 0)),
            pl.BlockSpec((1, D_HALF, D_FEAT), lambda i: (1, 0, 0)),
            pl.BlockSpec((1, D_FEAT), lambda i: (0, 0)),
        ],
        out_specs=pl.BlockSpec((bm, D_FEAT), lambda i: (i, 0)),
        out_shape=jax.ShapeDtypeStruct((N_NODES, D_FEAT), jnp.float32),
    )(agg, agg, d0, d1, W_gnn.reshape(2, D_HALF, D_FEAT),
      W_gnn.reshape(2, D_HALF, D_FEAT), b_gnn.reshape(1, D_FEAT))

    lpad = _LPAD - N_LINK
    e0_p = jnp.concatenate([e0, jnp.zeros((lpad,), jnp.int32)]).reshape(_NBLK3, _BLK3)
    e1_p = jnp.concatenate([e1, jnp.zeros((lpad,), jnp.int32)]).reshape(_NBLK3, _BLK3)
    epair = jnp.concatenate([e0_p, e1_p], axis=1)
    hi, hj = _make_p3()(h, epair)

    bl = 2048
    logits = pl.pallas_call(
        _mlp,
        grid=(_LPAD // bl,),
        in_specs=[
            pl.BlockSpec((bl, D_FEAT), lambda i: (i, 0)),
            pl.BlockSpec((bl, D_FEAT), lambda i: (i, 0)),
            pl.BlockSpec((D_FEAT, D_HID), lambda i: (0, 0)),
            pl.BlockSpec((1, D_HID), lambda i: (0, 0)),
            pl.BlockSpec((1, D_HID), lambda i: (0, 0)),
            pl.BlockSpec((1, 1), lambda i: (0, 0)),
        ],
        out_specs=pl.BlockSpec((bl, 1), lambda i: (i, 0)),
        out_shape=jax.ShapeDtypeStruct((_LPAD, 1), jnp.float32),
    )(hi, hj, W1, b1.reshape(1, D_HID), W2.reshape(1, D_HID), b2.reshape(1, 1))

    return logits[:N_LINK, 0]


# speedup vs baseline: 1.1052x; 1.0042x over previous
import functools

import jax
import jax.numpy as jnp
from jax import lax
from jax.experimental import pallas as pl
from jax.experimental.pallas import tpu as pltpu
from jax.experimental.pallas import tpu_sc as plsc

N_NODES = 10000
D_FEAT = 256
D_HALF = 128
D_HID = 128
N_ADJ = 160000
N_LINK = 20000

NC = 2
NS = 16
NW = NC * NS

_KB1 = 80
_EPW = _KB1 * 128
_EPAD = NS * _EPW
_ROWS1 = 2 * _EPAD // 128
_ACC_ROWS = 10112
_CH = 40

_BLK3 = 64
_KB3 = 10
_PPW = _KB3 * _BLK3
_LPAD = NW * _PPW
_NBLK3 = _LPAD // _BLK3

def _p1_body(xi, srcr, dstr, z2, z1, o1, agg_out, deg0_out, deg1_out,
             acc_sh, deg_sh, src_v, dst_v, buf_a, buf_b, ones_v, deg_buf,
             sem_a, sem_b, sem_s, sem_d):
    c = lax.axis_index("c")
    s = lax.axis_index("s")
    w = c * NS + s

    pltpu.sync_copy(o1, ones_v)

    rz = _ACC_ROWS // NS
    pltpu.sync_copy(z2, buf_a)
    for k in range(4):
        pltpu.sync_copy(buf_a, acc_sh.at[pl.ds(s * rz + k * 128, 128)])
    pltpu.sync_copy(buf_a.at[pl.ds(0, rz - 512)],
                    acc_sh.at[pl.ds(s * rz + 512, rz - 512)])
    pltpu.sync_copy(z1, deg_buf.at[pl.ds(0, rz)])
    pltpu.sync_copy(deg_buf.at[pl.ds(0, rz)], deg_sh.at[pl.ds(s * rz, rz)])

    plsc.subcore_barrier()

    bufs = (buf_a, buf_b)
    sems = (sem_a, sem_b)

    def wait_gather(b_par):
        pltpu.make_async_copy(xi.at[src_v.at[0]], bufs[b_par], sems[b_par]).wait()

    def issue_scatter(b_par, b):
        pltpu.async_copy(bufs[b_par], acc_sh.at[dst_v.at[b]], sem_s, add=True)

    def wait_scatter(b_par):
        pltpu.make_async_copy(bufs[b_par], acc_sh.at[dst_v.at[0]], sem_s).wait()

    for hp in range(_KB1 // _CH):
        base = w * _KB1 + hp * _CH
        pltpu.sync_copy(srcr.at[pl.ds(base, _CH)], src_v)
        pltpu.sync_copy(dstr.at[pl.ds(base, _CH)], dst_v)
        deg_on = c == hp

        def deg_scatter(b):
            @pl.when(deg_on)
            def _():
                pltpu.async_copy(ones_v, deg_sh.at[dst_v.at[b]], sem_d,
                                 add=True)

        pltpu.async_copy(xi.at[src_v.at[0]], buf_a, sem_a)
        pltpu.async_copy(xi.at[src_v.at[1]], buf_b, sem_b)
        wait_gather(0)
        issue_scatter(0, 0)
        deg_scatter(0)

        def halfpass(i, carry):
            b1 = 2 * i + 1
            wait_scatter(0)
            pltpu.async_copy(xi.at[src_v.at[b1 + 1]], buf_a, sem_a)
            wait_gather(1)
            issue_scatter(1, b1)
            deg_scatter(b1)

            b2 = b1 + 1
            wait_scatter(1)
            pltpu.async_copy(xi.at[src_v.at[b2 + 1]], buf_b, sem_b)
            wait_gather(0)
            issue_scatter(0, b2)
            deg_scatter(b2)
            return carry

        lax.fori_loop(0, _CH // 2 - 1, halfpass, 0)

        wait_scatter(0)
        wait_gather(1)
        issue_scatter(1, _CH - 1)
        deg_scatter(_CH - 1)
        wait_scatter(1)

        @pl.when(deg_on)
        def _():
            for _i in range(_CH):
                pltpu.make_async_copy(ones_v, deg_sh.at[dst_v.at[0]], sem_d).wait()

    plsc.subcore_barrier()

    ro = _ACC_ROWS // NS
    pltpu.sync_copy(acc_sh.at[pl.ds(s * ro, ro)], agg_out.at[c, pl.ds(s * ro, ro)])

    pltpu.sync_copy(deg_sh.at[pl.ds(s * rz, rz)], deg_buf.at[pl.ds(0, rz)])

    @pl.when(c == 0)
    def _():
        pltpu.sync_copy(deg_buf.at[pl.ds(0, rz)], deg0_out.at[pl.ds(s * rz, rz)])

    @pl.when(c == 1)
    def _():
        pltpu.sync_copy(deg_buf.at[pl.ds(0, rz)], deg1_out.at[pl.ds(s * rz, rz)])


@functools.lru_cache(maxsize=None)
def _make_p1():
    mesh = plsc.VectorSubcoreMesh(
        core_axis_name="c", subcore_axis_name="s", num_cores=NC, num_subcores=NS)
    return pl.kernel(
        _p1_body,
        out_type=(jax.ShapeDtypeStruct((NC, _ACC_ROWS, D_HALF), jnp.float32),
                  jax.ShapeDtypeStruct((_ACC_ROWS,), jnp.float32),
                  jax.ShapeDtypeStruct((_ACC_ROWS,), jnp.float32)),
        mesh=mesh,
        scratch_types=[
            pltpu.VMEM_SHARED((_ACC_ROWS, D_HALF), jnp.float32),
            pltpu.VMEM_SHARED((_ACC_ROWS,), jnp.float32),
            pltpu.VMEM((_CH, 128), jnp.int32),
            pltpu.VMEM((_CH, 128), jnp.int32),
            pltpu.VMEM((128, D_HALF), jnp.float32),
            pltpu.VMEM((128, D_HALF), jnp.float32),
            pltpu.VMEM((128,), jnp.float32),
            pltpu.VMEM((_ACC_ROWS // NS,), jnp.float32),
            pltpu.SemaphoreType.DMA,
            pltpu.SemaphoreType.DMA,
            pltpu.SemaphoreType.DMA,
            pltpu.SemaphoreType.DMA,
        ])


def _p3_body(h, epr, hi_out, hj_out, eb_v, b0, b1, b2,
             g0, g1, g2, w0, w1, w2):
    c = lax.axis_index("c")
    s = lax.axis_index("s")
    w = c * NS + s
    row0 = w * _KB3
    base8 = (row0 // 8) * 8
    off = row0 - base8
    pltpu.sync_copy(epr.at[pl.ds(base8, 16)], eb_v)

    bufs = (b0, b1, b2)
    gs = (g0, g1, g2)
    ws = (w0, w1, w2)

    def ig(p, k):
        pltpu.async_copy(h.at[eb_v.at[off + k]], bufs[p], gs[p])

    def wg(p):
        pltpu.make_async_copy(h.at[eb_v.at[0]], bufs[p], gs[p]).wait()

    def iw(p, k):
        base = w * _PPW + k * _BLK3
        pltpu.async_copy(bufs[p].at[pl.ds(0, 64)],
                         hi_out.at[pl.ds(base, 64)], ws[p])
        pltpu.async_copy(bufs[p].at[pl.ds(64, 64)],
                         hj_out.at[pl.ds(base, 64)], ws[p])

    def ww(p):
        pltpu.make_async_copy(bufs[p].at[pl.ds(0, 64)],
                              hi_out.at[pl.ds(0, 64)], ws[p]).wait()
        pltpu.make_async_copy(bufs[p].at[pl.ds(64, 64)],
                              hj_out.at[pl.ds(0, 64)], ws[p]).wait()

    ig(0, 0)
    ig(1, 1)
    ig(2, 2)
    for k in range(_KB3):
        p = k % 3
        wg(p)
        iw(p, k)
        if k + 3 < _KB3:
            ww(p)
            ig(p, k + 3)
    ww((_KB3 - 3) % 3)
    ww((_KB3 - 2) % 3)
    ww((_KB3 - 1) % 3)


@functools.lru_cache(maxsize=None)
def _make_p3():
    mesh = plsc.VectorSubcoreMesh(
        core_axis_name="c", subcore_axis_name="s", num_cores=NC, num_subcores=NS)
    return pl.kernel(
        _p3_body,
        out_type=(jax.ShapeDtypeStruct((_LPAD, D_FEAT), jnp.float32),
                  jax.ShapeDtypeStruct((_LPAD, D_FEAT), jnp.float32)),
        mesh=mesh,
        scratch_types=[
            pltpu.VMEM((16, 128), jnp.int32),
            pltpu.VMEM((128, D_FEAT), jnp.float32),
            pltpu.VMEM((128, D_FEAT), jnp.float32),
            pltpu.VMEM((128, D_FEAT), jnp.float32),
            pltpu.SemaphoreType.DMA,
            pltpu.SemaphoreType.DMA,
            pltpu.SemaphoreType.DMA,
            pltpu.SemaphoreType.DMA,
            pltpu.SemaphoreType.DMA,
            pltpu.SemaphoreType.DMA,
        ])


def _gnn_mm(a0_ref, a1_ref, d0_ref, d1_ref, w0_ref, w1_ref, b_ref, out_ref):
    r = 1.0 / jnp.maximum(d0_ref[...] + d1_ref[...], 1.0)
    acc = jnp.dot(a0_ref[0] * r, w0_ref[0], preferred_element_type=jnp.float32)
    acc += jnp.dot(a1_ref[0] * r, w1_ref[0], preferred_element_type=jnp.float32)
    out_ref[...] = jnp.maximum(acc + b_ref[...], 0.0)


def _mlp(hi_ref, hj_ref, w1_ref, b1_ref, w2_ref, b2_ref, out_ref):
    z = hi_ref[...] * hj_ref[...]
    t = jnp.dot(z, w1_ref[...], preferred_element_type=jnp.float32) + b1_ref[...]
    t = jnp.maximum(t, 0.0)
    logit = jnp.sum(t * w2_ref[...], axis=1, keepdims=True) + b2_ref[...]
    out_ref[...] = 1.0 / (1.0 + jnp.exp(-logit))


def kernel(x, edges, adj, W_gnn, b_gnn, W1, b1, W2, b2):
    x = x.astype(jnp.float32)
    src = adj[0].astype(jnp.int32)
    dst = adj[1].astype(jnp.int32)
    e0 = edges[0].astype(jnp.int32)
    e1 = edges[1].astype(jnp.int32)

    xi = x.reshape(N_NODES, 2, D_HALF).reshape(2 * N_NODES, D_HALF)
    pad = _EPAD - N_ADJ
    src_p = jnp.concatenate([src, jnp.zeros((pad,), jnp.int32)])
    dst_p = jnp.concatenate([dst, jnp.full((pad,), N_NODES, jnp.int32)])
    src_all = jnp.concatenate([2 * src_p, 2 * src_p + 1]).reshape(_ROWS1, 128)
    dst_all = jnp.concatenate([dst_p, dst_p]).reshape(_ROWS1, 128)
    z2 = jnp.zeros((128, D_HALF), jnp.float32)
    z1 = jnp.zeros((_ACC_ROWS // NS,), jnp.float32)
    o1 = jnp.ones((128,), jnp.float32)

    agg, deg0, deg1 = _make_p1()(xi, src_all, dst_all, z2, z1, o1)
    d0 = deg0.reshape(_ACC_ROWS, 1)
    d1 = deg1.reshape(_ACC_ROWS, 1)

    bm = 2000
    h = pl.pallas_call(
        _gnn_mm,
        grid=(N_NODES // bm,),
        in_specs=[
            pl.BlockSpec((1, bm, D_HALF), lambda i: (0, i, 0)),
            pl.BlockSpec((1, bm, D_HALF), lambda i: (1, i, 0)),
            pl.BlockSpec((bm, 1), lambda i: (i, 0)),
            pl.BlockSpec((bm, 1), lambda i: (i, 0)),
            pl.BlockSpec((1, D_HALF, D_FEAT), lambda i: (0, 0, 0)),
            pl.BlockSpec((1, D_HALF, D_FEAT), lambda i: (1, 0, 0)),
            pl.BlockSpec((1, D_FEAT), lambda i: (0, 0)),
        ],
        out_specs=pl.BlockSpec((bm, D_FEAT), lambda i: (i, 0)),
        out_shape=jax.ShapeDtypeStruct((N_NODES, D_FEAT), jnp.float32),
    )(agg, agg, d0, d1, W_gnn.reshape(2, D_HALF, D_FEAT),
      W_gnn.reshape(2, D_HALF, D_FEAT), b_gnn.reshape(1, D_FEAT))

    lpad = _LPAD - N_LINK
    e0_p = jnp.concatenate([e0, jnp.zeros((lpad,), jnp.int32)]).reshape(_NBLK3, _BLK3)
    e1_p = jnp.concatenate([e1, jnp.zeros((lpad,), jnp.int32)]).reshape(_NBLK3, _BLK3)
    epair = jnp.concatenate([e0_p, e1_p], axis=1)
    hi, hj = _make_p3()(h, epair)

    bl = 4096
    logits = pl.pallas_call(
        _mlp,
        grid=(_LPAD // bl,),
        in_specs=[
            pl.BlockSpec((bl, D_FEAT), lambda i: (i, 0)),
            pl.BlockSpec((bl, D_FEAT), lambda i: (i, 0)),
            pl.BlockSpec((D_FEAT, D_HID), lambda i: (0, 0)),
            pl.BlockSpec((1, D_HID), lambda i: (0, 0)),
            pl.BlockSpec((1, D_HID), lambda i: (0, 0)),
            pl.BlockSpec((1, 1), lambda i: (0, 0)),
        ],
        out_specs=pl.BlockSpec((bl, 1), lambda i: (i, 0)),
        out_shape=jax.ShapeDtypeStruct((_LPAD, 1), jnp.float32),
    )(hi, hj, W1, b1.reshape(1, D_HID), W2.reshape(1, D_HID), b2.reshape(1, 1))

    return logits[:N_LINK, 0]
